# Initial kernel scaffold; baseline (speedup 1.0000x reference)
import jax
import jax.numpy as jnp
from jax.experimental import pallas as pl


def kernel(jacobian, npmi_matrix):
    raise NotImplementedError("write your pallas kernel here")


# speedup vs baseline: 4.6281x; 4.6281x over previous
import functools

import jax
import jax.numpy as jnp
from jax import lax
from jax.experimental import pallas as pl
from jax.experimental.pallas import tpu as pltpu
from jax.experimental.pallas import tpu_sc as plsc

K = 20
NPMI_LAMBDA = 0.7
NPMI_SCALE = 100.0

T = 20
V = 10000
C = 5
W = V // C
NT = T * C
NC, NS = 2, 16
NW = NC * NS
STEPS = (NT + NW - 1) // NW


def _prep_body(beta_ref, premul_ref, idx_ref, w_ref):
    beta = beta_ref[...]
    col = lax.broadcasted_iota(jnp.int32, (T, V), 1)
    work = beta
    mask_acc = jnp.zeros((T, V), jnp.float32)
    idxs = []
    vals = []
    for _ in range(K):
        m = jnp.max(work, axis=1, keepdims=True)
        ik = jnp.min(jnp.where(work == m, col, V), axis=1, keepdims=True)
        oh = col == ik
        mask_acc = mask_acc + jnp.where(oh, 1.0, 0.0)
        work = jnp.where(oh, -jnp.inf, work)
        idxs.append(ik)
        vals.append(m)

    vmax = vals[0]
    exps = [jnp.exp(v - vmax) for v in vals]
    esum = functools.reduce(lambda a, b: a + b, exps)
    ws = [e / esum for e in exps]

    e = jnp.exp(beta - vmax)
    s = e / jnp.sum(e, axis=1, keepdims=True)

    total = jnp.sum(mask_acc, axis=0, keepdims=True)
    div = (total - mask_acc) > 0.0
    premul_ref[...] = NPMI_SCALE * s * s * jnp.where(div, 0.6 + 0.8, 0.6)

    col32 = lax.broadcasted_iota(jnp.int32, (T, 32), 1)
    idx32 = jnp.full((T, 32), -1, jnp.int32)
    w32 = jnp.zeros((T, 32), jnp.float32)
    for k in range(K):
        idx32 = jnp.where(col32 == k, idxs[k], idx32)
        w32 = jnp.where(col32 == k, ws[k], w32)
    idx_ref[...] = idx32
    w_ref[...] = w32


@jax.jit
def _prep(beta):
    return pl.pallas_call(
        _prep_body,
        out_shape=[
            jax.ShapeDtypeStruct((T, V), jnp.float32),
            jax.ShapeDtypeStruct((T, 32), jnp.int32),
            jax.ShapeDtypeStruct((T, 32), jnp.float32),
        ],
    )(beta)


def _sc_body(npmi_hbm, idx_hbm, w_hbm, prem_hbm, parts_hbm,
             gi_v, idxr_v, w_v, buf_v, pm_v, o_v, sem):
    wid = lax.axis_index("s") * NC + lax.axis_index("c")
    lane = lax.broadcasted_iota(jnp.int32, (16,), 0)

    for step in range(STEPS):
        j = step * NW + wid

        @pl.when(j < NT)
        def _():
            t = j // C
            c = j % C
            pltpu.sync_copy(idx_hbm.at[t], idxr_v)
            pltpu.sync_copy(w_hbm.at[t], w_v)
            pltpu.sync_copy(prem_hbm.at[j], pm_v)

            i0 = idxr_v[pl.ds(0, 16)]
            i4 = idxr_v[pl.ds(4, 16)]
            gi_v[pl.ds(0, 16)] = i0 * C + c
            gi_v[pl.ds(4, 16)] = i4 * C + c
            pltpu.async_copy(npmi_hbm.at[gi_v], buf_v, sem).wait()

            i16 = idxr_v[pl.ds(16, 16)]
            p0 = i0 - c * W
            p1 = i16 - c * W
            v0 = (p0 >= 0) & (p0 < W)
            v1 = (p1 >= 0) & (p1 < W)
            pc0 = jnp.where(v0, p0, 0)
            pc1 = jnp.where(v1, p1, 0)
            ones = jnp.full((16,), 1.0, jnp.float32)
            plsc.store_scatter(buf_v, [lane, pc0], ones, mask=v0)
            plsc.store_scatter(buf_v, [jnp.where(v1, lane + 16, 0), pc1],
                               ones, mask=v1)

            w0 = w_v[pl.ds(0, 16)]
            w1 = w_v[pl.ds(16, 16)]
            wk = [jnp.sum(jnp.where(lane == k, w0, 0.0)) for k in range(16)]
            wk += [jnp.sum(jnp.where(lane == (k - 16), w1, 0.0))
                   for k in range(16, K)]

            def body(vb, carry):
                bv, av, mnv, mxv = carry
                off = vb * 16
                acc = wk[0] * buf_v[0, pl.ds(off, 16)]
                for k in range(1, K):
                    acc = acc + wk[k] * buf_v[k, pl.ds(off, 16)]
                p = pm_v[pl.ds(off, 16)]
                return (bv + p * acc, av + p,
                        jnp.minimum(mnv, acc), jnp.maximum(mxv, acc))

            zero = jnp.zeros((16,), jnp.float32)
            init = (zero, zero,
                    jnp.full((16,), jnp.inf, jnp.float32),
                    jnp.full((16,), -jnp.inf, jnp.float32))
            bv, av, mnv, mxv = lax.fori_loop(0, W // 16, body, init)

            bs = jnp.sum(bv)
            as_ = jnp.sum(av)
            mns = jnp.min(mnv)
            mxs = jnp.max(mxv)
            ov = (jnp.where(lane == 0, bs, 0.0)
                  + jnp.where(lane == 1, mns, 0.0)
                  + jnp.where(lane == 2, mxs, 0.0)
                  + jnp.where(lane == 3, as_, 0.0))
            o_v[...] = ov
            pltpu.sync_copy(o_v, parts_hbm.at[j])


@jax.jit
def _sc_gather(npmi_flat, idx32, w32, prem):
    mesh = plsc.VectorSubcoreMesh(core_axis_name="c", subcore_axis_name="s")
    run = functools.partial(
        pl.kernel,
        out_type=jax.ShapeDtypeStruct((NT, 16), jnp.float32),
        mesh=mesh,
        scratch_types=[
            pltpu.VMEM((K,), jnp.int32),
            pltpu.VMEM((32,), jnp.int32),
            pltpu.VMEM((32,), jnp.float32),
            pltpu.VMEM((K, W), jnp.float32),
            pltpu.VMEM((W,), jnp.float32),
            pltpu.VMEM((16,), jnp.float32),
            pltpu.SemaphoreType.DMA,
        ],
        compiler_params=pltpu.CompilerParams(use_tc_tiling_on_sc=False,
                                             needs_layout_passes=False),
    )(_sc_body)
    return run(npmi_flat, idx32, w32, prem)


def _fin_body(parts_ref, out_ref):
    p = parts_ref[...]
    lane = lax.broadcasted_iota(jnp.int32, (T, C * 16), 1) % 16
    b = jnp.sum(jnp.where(lane == 0, p, 0.0), axis=1, keepdims=True)
    mn = jnp.min(jnp.where(lane == 1, p, jnp.inf), axis=1, keepdims=True)
    mx = jnp.max(jnp.where(lane == 2, p, -jnp.inf), axis=1, keepdims=True)
    a = jnp.sum(jnp.where(lane == 3, p, 0.0), axis=1, keepdims=True)
    lt = a - (b - mn * a) / (mx - mn)
    out_ref[0, 0] = jnp.sum(lt)


@jax.jit
def _finalize(parts):
    return pl.pallas_call(
        _fin_body,
        out_shape=jax.ShapeDtypeStruct((1, 1), jnp.float32),
        out_specs=pl.BlockSpec(memory_space=pltpu.SMEM),
    )(parts)


def kernel(jacobian, npmi_matrix):
    beta = jacobian.T
    premul, idx32, w32 = _prep(beta)
    parts = _sc_gather(npmi_matrix.reshape(V * C, W), idx32, w32,
                       premul.reshape(NT, W))
    loss = _finalize(parts.reshape(T, C * 16))
    return loss[0, 0]


# speedup vs baseline: 29.8462x; 6.4490x over previous
import functools

import jax
import jax.numpy as jnp
from jax import lax
from jax.experimental import pallas as pl
from jax.experimental.pallas import tpu as pltpu
from jax.experimental.pallas import tpu_sc as plsc

K = 20
NPMI_LAMBDA = 0.7
NPMI_SCALE = 100.0

T = 20
V = 10000
VP = 10112
CW = 1664
C = 6
TAIL = V - C * CW
NT = T * C
NC, NS = 2, 16
NW = NC * NS
STEPS = (NT + NW - 1) // NW
NB = CW // 16
NEG = -1e30


def _prep_body(beta_ref, premul_ref, idx_ref, w_ref):
    beta = beta_ref[...]
    col = lax.broadcasted_iota(jnp.int32, (T, VP), 1)
    work = beta
    mask_acc = jnp.zeros((T, VP), jnp.float32)
    idxs = []
    vals = []
    for _ in range(K):
        m = jnp.max(work, axis=1, keepdims=True)
        ik = jnp.min(jnp.where(work == m, col, VP), axis=1, keepdims=True)
        oh = col == ik
        mask_acc = mask_acc + jnp.where(oh, 1.0, 0.0)
        work = jnp.where(oh, -jnp.inf, work)
        idxs.append(ik)
        vals.append(m)

    vmax = vals[0]
    exps = [jnp.exp(v - vmax) for v in vals]
    esum = functools.reduce(lambda a, b: a + b, exps)
    ws = [e / esum for e in exps]

    e = jnp.exp(beta - vmax)
    s = e / jnp.sum(e, axis=1, keepdims=True)

    total = jnp.sum(mask_acc, axis=0, keepdims=True)
    div = (total - mask_acc) > 0.0
    premul_ref[...] = NPMI_SCALE * s * s * jnp.where(div, 1.4, 0.6)

    col128 = lax.broadcasted_iota(jnp.int32, (T, 128), 1)
    idx128 = jnp.full((T, 128), -1, jnp.int32)
    w128 = jnp.zeros((T, 128), jnp.float32)
    for k in range(K):
        idx128 = jnp.where(col128 == k, idxs[k], idx128)
        w128 = jnp.where(col128 == k, ws[k], w128)
    idx_ref[...] = idx128
    w_ref[...] = w128


@jax.jit
def _prep(beta_padded):
    return pl.pallas_call(
        _prep_body,
        out_shape=[
            jax.ShapeDtypeStruct((T, VP), jnp.float32),
            jax.ShapeDtypeStruct((T, 128), jnp.int32),
            jax.ShapeDtypeStruct((T, 128), jnp.float32),
        ],
    )(beta_padded)


def _sc_body(npmi_hbm, tslab_hbm, idxf_hbm, wf_hbm, pmf_hbm, parts_hbm,
             gi_v, iv_v, wv_v, buf_v, pm_v, corr_v, db_v, tail_v, pmt_v, o_v,
             sem, semd, semt):
    wid = lax.axis_index("s") * NC + lax.axis_index("c")
    lane = lax.broadcasted_iota(jnp.int32, (16,), 0)
    zero16 = jnp.zeros((16,), jnp.float32)

    for step in range(STEPS):
        j = step * NW + wid

        @pl.when(j < NT)
        def _():
            t = j // C
            c = j % C
            col0 = c * CW
            pltpu.sync_copy(idxf_hbm.at[pl.ds(t * 128, 128)], iv_v)
            pltpu.sync_copy(wf_hbm.at[pl.ds(t * 128, 128)], wv_v)

            i0 = iv_v[pl.ds(0, 16)]
            i16 = iv_v[pl.ds(16, 16)]
            w0 = wv_v[pl.ds(0, 16)]
            w1 = wv_v[pl.ds(16, 16)]
            gi_v[pl.ds(0, 16)] = i0
            m1 = lane < (K - 16)
            i16c0 = jnp.where(m1, i16, 0)
            m8 = lane < 8
            plsc.store_scatter(gi_v, [jnp.where(m8, lane + 16, 23)], i16c0,
                               mask=m8)

            iks = [jnp.sum(jnp.where(lane == k, i0, 0)) for k in range(16)]
            iks += [jnp.sum(jnp.where(lane == (k - 16), i16, 0))
                    for k in range(16, K)]
            wks = [jnp.sum(jnp.where(lane == k, w0, 0.0)) for k in range(16)]
            wks += [jnp.sum(jnp.where(lane == (k - 16), w1, 0.0))
                    for k in range(16, K)]

            main_cp = pltpu.async_copy(
                npmi_hbm.at[gi_v, pl.ds(col0, CW)], buf_v, sem)

            diag_cps = []
            for k in range(K):
                co = pl.multiple_of((iks[k] // 128) * 128, 128)
                diag_cps.append(pltpu.async_copy(
                    npmi_hbm.at[iks[k], pl.ds(co, 128)],
                    db_v.at[pl.ds(k * 128, 128)], semd))

            pltpu.sync_copy(pmf_hbm.at[pl.ds(t * VP + col0, CW)], pm_v)

            for i in range(NB):
                corr_v[pl.ds(i * 16, 16)] = zero16

            for cp in diag_cps:
                cp.wait()

            g0 = lane * 128 + lax.rem(i0, 128)
            d0 = plsc.load_gather(db_v, [g0])
            i16c = jnp.where(m1, i16, 0)
            g1 = (lane + 16) * 128 + lax.rem(i16c, 128)
            d1 = plsc.load_gather(db_v, [jnp.where(m1, g1, 0)])
            delta0 = w0 * (1.0 - d0)
            delta1 = jnp.where(m1, w1 * (1.0 - d1), 0.0)
            p0 = i0 - col0
            p1 = i16c - col0
            in0 = (p0 >= 0) & (p0 < CW)
            in1 = m1 & (p1 >= 0) & (p1 < CW)
            plsc.store_scatter(corr_v, [jnp.where(in0, p0, 0)], delta0,
                               mask=in0)
            plsc.store_scatter(corr_v, [jnp.where(in1, p1, 0)], delta1,
                               mask=in1)

            main_cp.wait()

            def body(vb, carry):
                bv, av, mnv, mxv = carry
                off = vb * 16
                acc = corr_v[pl.ds(off, 16)]
                for k in range(K):
                    acc = acc + wks[k] * buf_v[k, pl.ds(off, 16)]
                p = pm_v[pl.ds(off, 16)]
                return (bv + p * acc, av + p,
                        jnp.minimum(mnv, acc), jnp.maximum(mxv, acc))

            init = (zero16, zero16,
                    jnp.full((16,), jnp.inf, jnp.float32),
                    jnp.full((16,), -jnp.inf, jnp.float32))
            bv, av, mnv, mxv = lax.fori_loop(0, NB, body, init)

            @pl.when(c == C - 1)
            def _tail():
                tail_cp = pltpu.async_copy(tslab_hbm.at[gi_v], tail_v, semt)
                pltpu.sync_copy(pmf_hbm.at[pl.ds(t * VP + C * CW, 128)],
                                pmt_v)
                tail_cp.wait()
                mt = zero16
                for k in range(K):
                    mt = mt + wks[k] * tail_v[k, pl.ds(0, 16)]
                corr_v[pl.ds(0, 16)] = zero16
                pt0 = i0 - C * CW
                pt1 = i16c - C * CW
                it0 = (pt0 >= 0) & (pt0 < TAIL)
                it1 = m1 & (pt1 >= 0) & (pt1 < TAIL)
                plsc.store_scatter(corr_v, [jnp.where(it0, pt0, 0)], delta0,
                                   mask=it0)
                plsc.store_scatter(corr_v, [jnp.where(it1, pt1, 0)], delta1,
                                   mask=it1)
                mt2 = mt + corr_v[pl.ds(0, 16)]
                pmt = pmt_v[pl.ds(0, 16)]
                bs = jnp.sum(bv + pmt * mt2)
                as_ = jnp.sum(av + pmt)
                mns = jnp.min(jnp.minimum(mnv, mt2))
                mxs = jnp.max(jnp.maximum(mxv, mt2))
                ov = (jnp.where(lane == 0, bs, 0.0)
                      + jnp.where(lane == 1, mns, 0.0)
                      + jnp.where(lane == 2, mxs, 0.0)
                      + jnp.where(lane == 3, as_, 0.0))
                o_v[pl.ds(0, 16)] = ov
                pltpu.sync_copy(o_v, parts_hbm.at[pl.ds(j * 128, 128)])

            @pl.when(c != C - 1)
            def _notail():
                bs = jnp.sum(bv)
                as_ = jnp.sum(av)
                mns = jnp.min(mnv)
                mxs = jnp.max(mxv)
                ov = (jnp.where(lane == 0, bs, 0.0)
                      + jnp.where(lane == 1, mns, 0.0)
                      + jnp.where(lane == 2, mxs, 0.0)
                      + jnp.where(lane == 3, as_, 0.0))
                o_v[pl.ds(0, 16)] = ov
                pltpu.sync_copy(o_v, parts_hbm.at[pl.ds(j * 128, 128)])


@jax.jit
def _sc_gather(npmi, tslab, idxf, wf, pmf):
    mesh = plsc.VectorSubcoreMesh(core_axis_name="c", subcore_axis_name="s")
    run = functools.partial(
        pl.kernel,
        out_type=jax.ShapeDtypeStruct((NT * 128,), jnp.float32),
        mesh=mesh,
        scratch_types=[
            pltpu.VMEM((24,), jnp.int32),
            pltpu.VMEM((128,), jnp.int32),
            pltpu.VMEM((128,), jnp.float32),
            pltpu.VMEM((24, CW), jnp.float32),
            pltpu.VMEM((CW,), jnp.float32),
            pltpu.VMEM((CW,), jnp.float32),
            pltpu.VMEM((K * 128,), jnp.float32),
            pltpu.VMEM((24, 128), jnp.float32),
            pltpu.VMEM((128,), jnp.float32),
            pltpu.VMEM((128,), jnp.float32),
            pltpu.SemaphoreType.DMA,
            pltpu.SemaphoreType.DMA,
            pltpu.SemaphoreType.DMA,
        ],
        compiler_params=pltpu.CompilerParams(use_tc_tiling_on_sc=True,
                                             needs_layout_passes=False),
    )(_sc_body)
    return run(npmi, tslab, idxf, wf, pmf)


def _fin_body(parts_ref, out_ref):
    p = parts_ref[...]
    lane = lax.broadcasted_iota(jnp.int32, (T, C * 128), 1) % 128
    b = jnp.sum(jnp.where(lane == 0, p, 0.0), axis=1, keepdims=True)
    mn = jnp.min(jnp.where(lane == 1, p, jnp.inf), axis=1, keepdims=True)
    mx = jnp.max(jnp.where(lane == 2, p, -jnp.inf), axis=1, keepdims=True)
    a = jnp.sum(jnp.where(lane == 3, p, 0.0), axis=1, keepdims=True)
    lt = a - (b - mn * a) / (mx - mn)
    out_ref[0, 0] = jnp.sum(lt)


@jax.jit
def _finalize(parts):
    return pl.pallas_call(
        _fin_body,
        out_shape=jax.ShapeDtypeStruct((1, 1), jnp.float32),
        out_specs=pl.BlockSpec(memory_space=pltpu.SMEM),
    )(parts)


def kernel(jacobian, npmi_matrix):
    beta = jacobian.T
    beta_p = jnp.pad(beta, ((0, 0), (0, VP - V)), constant_values=NEG)
    premul, idx128, w128 = _prep(beta_p)
    tslab = jnp.pad(lax.slice(npmi_matrix, (0, C * CW), (V, V)),
                    ((0, 0), (0, 128 - TAIL)))
    parts = _sc_gather(npmi_matrix, tslab, idx128.reshape(T * 128),
                       w128.reshape(T * 128), premul.reshape(T * VP))
    loss = _finalize(parts.reshape(T, C * 128))
    return loss[0, 0]


# speedup vs baseline: 33.1708x; 1.1114x over previous
import functools

import jax
import jax.numpy as jnp
from jax import lax
from jax.experimental import pallas as pl
from jax.experimental.pallas import tpu as pltpu
from jax.experimental.pallas import tpu_sc as plsc

K = 20
NPMI_LAMBDA = 0.7
NPMI_SCALE = 100.0

T = 20
V = 10000
VP = 10112
CW = 1664
C = 6
TAIL = V - C * CW
NT = T * C
NC, NS = 2, 16
NW = NC * NS
STEPS = (NT + NW - 1) // NW
NB = CW // 16
NEG = -1e30


def _prep_body(beta_ref, premul_ref, idx_ref, w_ref):
    beta = beta_ref[...]
    col = lax.broadcasted_iota(jnp.int32, (T, VP), 1)
    work = beta
    mask_acc = jnp.zeros((T, VP), jnp.float32)
    idxs = []
    vals = []
    for _ in range(K):
        m = jnp.max(work, axis=1, keepdims=True)
        ik = jnp.min(jnp.where(work == m, col, VP), axis=1, keepdims=True)
        oh = col == ik
        mask_acc = mask_acc + jnp.where(oh, 1.0, 0.0)
        work = jnp.where(oh, -jnp.inf, work)
        idxs.append(ik)
        vals.append(m)

    vmax = vals[0]
    exps = [jnp.exp(v - vmax) for v in vals]
    esum = functools.reduce(lambda a, b: a + b, exps)
    ws = [e / esum for e in exps]

    e = jnp.exp(beta - vmax)
    s = e / jnp.sum(e, axis=1, keepdims=True)

    total = jnp.sum(mask_acc, axis=0, keepdims=True)
    div = (total - mask_acc) > 0.0
    premul_ref[...] = NPMI_SCALE * s * s * jnp.where(div, 1.4, 0.6)

    col128 = lax.broadcasted_iota(jnp.int32, (T, 128), 1)
    idx128 = jnp.full((T, 128), -1, jnp.int32)
    w128 = jnp.zeros((T, 128), jnp.float32)
    for k in range(K):
        idx128 = jnp.where(col128 == k, idxs[k], idx128)
        w128 = jnp.where(col128 == k, ws[k], w128)
    idx_ref[...] = idx128
    w_ref[...] = w128


@jax.jit
def _prep(beta_padded):
    return pl.pallas_call(
        _prep_body,
        out_shape=[
            jax.ShapeDtypeStruct((T, VP), jnp.float32),
            jax.ShapeDtypeStruct((T, 128), jnp.int32),
            jax.ShapeDtypeStruct((T, 128), jnp.float32),
        ],
    )(beta_padded)


def _sc_body(npmi_hbm, tslab_hbm, idxf_hbm, wf_hbm, pmf_hbm, parts_hbm,
             gi0_v, gi1_v, iv0_v, iv1_v, wv0_v, wv1_v, buf0_v, buf1_v,
             b20_v, b21_v, pm0_v, pm1_v, corr_v, db_v, tail_v, t2_v, pmt_v,
             o_v, sem0, sem1, semd, semt):
    wid = lax.axis_index("s") * NC + lax.axis_index("c")
    lane = lax.broadcasted_iota(jnp.int32, (16,), 0)
    zero16 = jnp.zeros((16,), jnp.float32)
    m1 = lane < (K - 16)
    bufs = [(gi0_v, iv0_v, wv0_v, buf0_v, b20_v, pm0_v, sem0),
            (gi1_v, iv1_v, wv1_v, buf1_v, b21_v, pm1_v, sem1)]

    def prefetch(step):
        jn = step * NW + wid
        gi, iv, wv, buf, b2, pm, ps = bufs[step % 2]

        @pl.when(jn < NT)
        def _():
            tn = jn // C
            cn = jn % C
            coln = cn * CW
            pltpu.sync_copy(idxf_hbm.at[pl.ds(tn * 128, 128)], iv)
            pltpu.sync_copy(wf_hbm.at[pl.ds(tn * 128, 128)], wv)
            i0n = iv[pl.ds(0, 16)]
            i16n = iv[pl.ds(16, 16)]
            gi[pl.ds(0, 16)] = i0n
            pltpu.async_copy(npmi_hbm.at[gi, pl.ds(coln, CW)], buf, ps)
            i16s = [jnp.sum(jnp.where(lane == k2, i16n, 0))
                    for k2 in range(K - 16)]
            for k2 in range(K - 16):
                pltpu.async_copy(npmi_hbm.at[i16s[k2], pl.ds(coln, CW)],
                                 b2.at[pl.ds(k2 * CW, CW)], ps)
            pltpu.async_copy(pmf_hbm.at[pl.ds(tn * VP + coln, CW)], pm, ps)

    prefetch(0)
    for step in range(STEPS):
        j = step * NW + wid
        gi, iv, wv, buf, b2, pm, ps = bufs[step % 2]
        if step + 1 < STEPS:
            prefetch(step + 1)

        @pl.when(j < NT)
        def _():
            t = j // C
            c = j % C
            col0 = c * CW

            i0 = iv[pl.ds(0, 16)]
            i16 = iv[pl.ds(16, 16)]
            w0 = wv[pl.ds(0, 16)]
            w1 = wv[pl.ds(16, 16)]

            iks = [jnp.sum(jnp.where(lane == k, i0, 0)) for k in range(16)]
            iks += [jnp.sum(jnp.where(lane == (k - 16), i16, 0))
                    for k in range(16, K)]
            wks = [jnp.sum(jnp.where(lane == k, w0, 0.0)) for k in range(16)]
            wks += [jnp.sum(jnp.where(lane == (k - 16), w1, 0.0))
                    for k in range(16, K)]

            diag_cps = []
            for k in range(K):
                co = pl.multiple_of((iks[k] // 128) * 128, 128)
                diag_cps.append(pltpu.async_copy(
                    npmi_hbm.at[iks[k], pl.ds(co, 128)],
                    db_v.at[pl.ds(k * 128, 128)], semd))

            for i in range(NB):
                corr_v[pl.ds(i * 16, 16)] = zero16

            for cp in diag_cps:
                cp.wait()

            g0 = lane * 128 + lax.rem(i0, 128)
            d0 = plsc.load_gather(db_v, [g0])
            i16c = jnp.where(m1, i16, 0)
            g1 = (lane + 16) * 128 + lax.rem(i16c, 128)
            d1 = plsc.load_gather(db_v, [jnp.where(m1, g1, 0)])
            delta0 = w0 * (1.0 - d0)
            delta1 = jnp.where(m1, w1 * (1.0 - d1), 0.0)
            p0 = i0 - col0
            p1 = i16c - col0
            in0 = (p0 >= 0) & (p0 < CW)
            in1 = m1 & (p1 >= 0) & (p1 < CW)
            plsc.store_scatter(corr_v, [jnp.where(in0, p0, 0)], delta0,
                               mask=in0)
            plsc.store_scatter(corr_v, [jnp.where(in1, p1, 0)], delta1,
                               mask=in1)

            pltpu.make_async_copy(
                npmi_hbm.at[gi, pl.ds(col0, CW)], buf, ps).wait()
            for k2 in range(K - 16):
                pltpu.make_async_copy(
                    npmi_hbm.at[iks[16 + k2], pl.ds(col0, CW)],
                    b2.at[pl.ds(k2 * CW, CW)], ps).wait()
            pltpu.make_async_copy(
                pmf_hbm.at[pl.ds(t * VP + col0, CW)], pm, ps).wait()

            def body(vb, carry):
                bv, av, mnv, mxv = carry
                off = vb * 16
                acc = corr_v[pl.ds(off, 16)]
                for k in range(16):
                    acc = acc + wks[k] * buf[k, pl.ds(off, 16)]
                for k2 in range(K - 16):
                    acc = acc + wks[16 + k2] * b2[pl.ds(k2 * CW + off, 16)]
                p = pm[pl.ds(off, 16)]
                return (bv + p * acc, av + p,
                        jnp.minimum(mnv, acc), jnp.maximum(mxv, acc))

            init = (zero16, zero16,
                    jnp.full((16,), jnp.inf, jnp.float32),
                    jnp.full((16,), -jnp.inf, jnp.float32))
            bv, av, mnv, mxv = lax.fori_loop(0, NB, body, init)

            @pl.when(c == C - 1)
            def _tail():
                tail_cp = pltpu.async_copy(tslab_hbm.at[gi], tail_v, semt)
                t2_cps = [pltpu.async_copy(
                    tslab_hbm.at[iks[16 + k2], pl.ds(0, 128)],
                    t2_v.at[pl.ds(k2 * 128, 128)], semt)
                    for k2 in range(K - 16)]
                pltpu.sync_copy(pmf_hbm.at[pl.ds(t * VP + C * CW, 128)],
                                pmt_v)
                tail_cp.wait()
                for cp in t2_cps:
                    cp.wait()
                mt = zero16
                for k in range(16):
                    mt = mt + wks[k] * tail_v[k, pl.ds(0, 16)]
                for k2 in range(K - 16):
                    mt = mt + wks[16 + k2] * t2_v[pl.ds(k2 * 128, 16)]
                corr_v[pl.ds(0, 16)] = zero16
                pt0 = i0 - C * CW
                pt1 = i16c - C * CW
                it0 = (pt0 >= 0) & (pt0 < TAIL)
                it1 = m1 & (pt1 >= 0) & (pt1 < TAIL)
                plsc.store_scatter(corr_v, [jnp.where(it0, pt0, 0)], delta0,
                                   mask=it0)
                plsc.store_scatter(corr_v, [jnp.where(it1, pt1, 0)], delta1,
                                   mask=it1)
                mt2 = mt + corr_v[pl.ds(0, 16)]
                pmt = pmt_v[pl.ds(0, 16)]
                bs = jnp.sum(bv + pmt * mt2)
                as_ = jnp.sum(av + pmt)
                mns = jnp.min(jnp.minimum(mnv, mt2))
                mxs = jnp.max(jnp.maximum(mxv, mt2))
                ov = (jnp.where(lane == 0, bs, 0.0)
                      + jnp.where(lane == 1, mns, 0.0)
                      + jnp.where(lane == 2, mxs, 0.0)
                      + jnp.where(lane == 3, as_, 0.0))
                o_v[pl.ds(0, 16)] = ov
                pltpu.sync_copy(o_v, parts_hbm.at[pl.ds(j * 128, 128)])

            @pl.when(c != C - 1)
            def _notail():
                bs = jnp.sum(bv)
                as_ = jnp.sum(av)
                mns = jnp.min(mnv)
                mxs = jnp.max(mxv)
                ov = (jnp.where(lane == 0, bs, 0.0)
                      + jnp.where(lane == 1, mns, 0.0)
                      + jnp.where(lane == 2, mxs, 0.0)
                      + jnp.where(lane == 3, as_, 0.0))
                o_v[pl.ds(0, 16)] = ov
                pltpu.sync_copy(o_v, parts_hbm.at[pl.ds(j * 128, 128)])


@jax.jit
def _sc_gather(npmi, tslab, idxf, wf, pmf):
    mesh = plsc.VectorSubcoreMesh(core_axis_name="c", subcore_axis_name="s")
    dbl = lambda shape, dt: [pltpu.VMEM(shape, dt), pltpu.VMEM(shape, dt)]
    run = functools.partial(
        pl.kernel,
        out_type=jax.ShapeDtypeStruct((NT * 128,), jnp.float32),
        mesh=mesh,
        scratch_types=(
            dbl((16,), jnp.int32)
            + dbl((128,), jnp.int32)
            + dbl((128,), jnp.float32)
            + dbl((16, CW), jnp.float32)
            + dbl(((K - 16) * CW,), jnp.float32)
            + dbl((CW,), jnp.float32)
            + [
                pltpu.VMEM((CW,), jnp.float32),
                pltpu.VMEM((K * 128,), jnp.float32),
                pltpu.VMEM((16, 128), jnp.float32),
                pltpu.VMEM(((K - 16) * 128,), jnp.float32),
                pltpu.VMEM((128,), jnp.float32),
                pltpu.VMEM((128,), jnp.float32),
                pltpu.SemaphoreType.DMA,
                pltpu.SemaphoreType.DMA,
                pltpu.SemaphoreType.DMA,
                pltpu.SemaphoreType.DMA,
            ]
        ),
        compiler_params=pltpu.CompilerParams(use_tc_tiling_on_sc=True,
                                             needs_layout_passes=False),
    )(_sc_body)
    return run(npmi, tslab, idxf, wf, pmf)


def _fin_body(parts_ref, out_ref):
    p = parts_ref[...]
    lane = lax.broadcasted_iota(jnp.int32, (T, C * 128), 1) % 128
    b = jnp.sum(jnp.where(lane == 0, p, 0.0), axis=1, keepdims=True)
    mn = jnp.min(jnp.where(lane == 1, p, jnp.inf), axis=1, keepdims=True)
    mx = jnp.max(jnp.where(lane == 2, p, -jnp.inf), axis=1, keepdims=True)
    a = jnp.sum(jnp.where(lane == 3, p, 0.0), axis=1, keepdims=True)
    lt = a - (b - mn * a) / (mx - mn)
    out_ref[0, 0] = jnp.sum(lt)


@jax.jit
def _finalize(parts):
    return pl.pallas_call(
        _fin_body,
        out_shape=jax.ShapeDtypeStruct((1, 1), jnp.float32),
        out_specs=pl.BlockSpec(memory_space=pltpu.SMEM),
    )(parts)


def kernel(jacobian, npmi_matrix):
    beta = jacobian.T
    beta_p = jnp.pad(beta, ((0, 0), (0, VP - V)), constant_values=NEG)
    premul, idx128, w128 = _prep(beta_p)
    tslab = jnp.pad(lax.slice(npmi_matrix, (0, C * CW), (V, V)),
                    ((0, 0), (0, 128 - TAIL)))
    parts = _sc_gather(npmi_matrix, tslab, idx128.reshape(T * 128),
                       w128.reshape(T * 128), premul.reshape(T * VP))
    loss = _finalize(parts.reshape(T, C * 128))
    return loss[0, 0]


# speedup vs baseline: 33.3599x; 1.0057x over previous
import functools

import jax
import jax.numpy as jnp
from jax import lax
from jax.experimental import pallas as pl
from jax.experimental.pallas import tpu as pltpu
from jax.experimental.pallas import tpu_sc as plsc

K = 20
NPMI_LAMBDA = 0.7
NPMI_SCALE = 100.0

T = 20
V = 10000
VP = 10112
CW = 1664
C = 6
TAIL = V - C * CW
NT = T * C
NC, NS = 2, 16
NW = NC * NS
STEPS = (NT + NW - 1) // NW
NB = CW // 16
NEG = -1e30


def _prep_body(beta_ref, premul_ref, idx_ref, w_ref):
    beta = beta_ref[...]
    col = lax.broadcasted_iota(jnp.int32, (T, VP), 1)
    work = beta
    mask_acc = jnp.zeros((T, VP), jnp.float32)
    idxs = []
    vals = []
    for _ in range(K):
        m = jnp.max(work, axis=1, keepdims=True)
        ik = jnp.min(jnp.where(work == m, col, VP), axis=1, keepdims=True)
        oh = col == ik
        mask_acc = mask_acc + jnp.where(oh, 1.0, 0.0)
        work = jnp.where(oh, -jnp.inf, work)
        idxs.append(ik)
        vals.append(m)

    vmax = vals[0]
    exps = [jnp.exp(v - vmax) for v in vals]
    esum = functools.reduce(lambda a, b: a + b, exps)
    ws = [e / esum for e in exps]

    e = jnp.exp(beta - vmax)
    s = e / jnp.sum(e, axis=1, keepdims=True)

    total = jnp.sum(mask_acc, axis=0, keepdims=True)
    div = (total - mask_acc) > 0.0
    premul_ref[...] = NPMI_SCALE * s * s * jnp.where(div, 1.4, 0.6)

    col128 = lax.broadcasted_iota(jnp.int32, (T, 128), 1)
    idx128 = jnp.full((T, 128), -1, jnp.int32)
    w128 = jnp.zeros((T, 128), jnp.float32)
    for k in range(K):
        idx128 = jnp.where(col128 == k, idxs[k], idx128)
        w128 = jnp.where(col128 == k, ws[k], w128)
    idx_ref[...] = idx128
    w_ref[...] = w128


@jax.jit
def _prep(beta_padded):
    return pl.pallas_call(
        _prep_body,
        out_shape=[
            jax.ShapeDtypeStruct((T, VP), jnp.float32),
            jax.ShapeDtypeStruct((T, 128), jnp.int32),
            jax.ShapeDtypeStruct((T, 128), jnp.float32),
        ],
    )(beta_padded)


def _sc_body(npmi_hbm, tslab_hbm, idxf_hbm, wf_hbm, pmf_hbm, parts_hbm,
             gi0_v, gi1_v, iv0_v, iv1_v, wv0_v, wv1_v, buf0_v, buf1_v,
             b20_v, b21_v, pm0_v, pm1_v, corr_v, db_v, tail_v, t2_v, pmt_v,
             o_v, sem0, sem1, semd, semt):
    wid = lax.axis_index("s") * NC + lax.axis_index("c")
    lane = lax.broadcasted_iota(jnp.int32, (16,), 0)
    zero16 = jnp.zeros((16,), jnp.float32)
    m1 = lane < (K - 16)
    bufs = [(gi0_v, iv0_v, wv0_v, buf0_v, b20_v, pm0_v, sem0),
            (gi1_v, iv1_v, wv1_v, buf1_v, b21_v, pm1_v, sem1)]

    def prefetch(step):
        jn = step * NW + wid
        gi, iv, wv, buf, b2, pm, ps = bufs[step % 2]

        @pl.when(jn < NT)
        def _():
            tn = jn // C
            cn = jn % C
            coln = cn * CW
            pltpu.sync_copy(idxf_hbm.at[pl.ds(tn * 128, 128)], iv)
            pltpu.sync_copy(wf_hbm.at[pl.ds(tn * 128, 128)], wv)
            i0n = iv[pl.ds(0, 16)]
            i16n = iv[pl.ds(16, 16)]
            gi[pl.ds(0, 16)] = i0n
            pltpu.async_copy(npmi_hbm.at[gi, pl.ds(coln, CW)], buf, ps)
            i16s = [jnp.sum(jnp.where(lane == k2, i16n, 0))
                    for k2 in range(K - 16)]
            for k2 in range(K - 16):
                pltpu.async_copy(npmi_hbm.at[i16s[k2], pl.ds(coln, CW)],
                                 b2.at[pl.ds(k2 * CW, CW)], ps)
            pltpu.async_copy(pmf_hbm.at[pl.ds(tn * VP + coln, CW)], pm, ps)

    prefetch(0)
    for step in range(STEPS):
        j = step * NW + wid
        gi, iv, wv, buf, b2, pm, ps = bufs[step % 2]
        if step + 1 < STEPS:
            prefetch(step + 1)

        @pl.when(j < NT)
        def _():
            t = j // C
            c = j % C
            col0 = c * CW

            i0 = iv[pl.ds(0, 16)]
            i16 = iv[pl.ds(16, 16)]
            w0 = wv[pl.ds(0, 16)]
            w1 = wv[pl.ds(16, 16)]

            iks = [jnp.sum(jnp.where(lane == k, i0, 0)) for k in range(16)]
            iks += [jnp.sum(jnp.where(lane == (k - 16), i16, 0))
                    for k in range(16, K)]
            wks = [jnp.sum(jnp.where(lane == k, w0, 0.0)) for k in range(16)]
            wks += [jnp.sum(jnp.where(lane == (k - 16), w1, 0.0))
                    for k in range(16, K)]

            diag_cps = []
            for k in range(K):
                co = pl.multiple_of((iks[k] // 128) * 128, 128)
                diag_cps.append(pltpu.async_copy(
                    npmi_hbm.at[iks[k], pl.ds(co, 128)],
                    db_v.at[pl.ds(k * 128, 128)], semd))

            for i in range(NB):
                corr_v[pl.ds(i * 16, 16)] = zero16

            for cp in diag_cps:
                cp.wait()

            g0 = lane * 128 + lax.rem(i0, 128)
            d0 = plsc.load_gather(db_v, [g0])
            i16c = jnp.where(m1, i16, 0)
            g1 = (lane + 16) * 128 + lax.rem(i16c, 128)
            d1 = plsc.load_gather(db_v, [jnp.where(m1, g1, 0)])
            delta0 = w0 * (1.0 - d0)
            delta1 = jnp.where(m1, w1 * (1.0 - d1), 0.0)
            p0 = i0 - col0
            p1 = i16c - col0
            in0 = (p0 >= 0) & (p0 < CW)
            in1 = m1 & (p1 >= 0) & (p1 < CW)
            plsc.store_scatter(corr_v, [jnp.where(in0, p0, 0)], delta0,
                               mask=in0)
            plsc.store_scatter(corr_v, [jnp.where(in1, p1, 0)], delta1,
                               mask=in1)

            pltpu.make_async_copy(
                npmi_hbm.at[gi, pl.ds(col0, CW)], buf, ps).wait()
            for k2 in range(K - 16):
                pltpu.make_async_copy(
                    npmi_hbm.at[iks[16 + k2], pl.ds(col0, CW)],
                    b2.at[pl.ds(k2 * CW, CW)], ps).wait()
            pltpu.make_async_copy(
                pmf_hbm.at[pl.ds(t * VP + col0, CW)], pm, ps).wait()

            def body(vb, carry):
                bv, av, mnv, mxv = carry
                off = vb * 16
                accs = [corr_v[pl.ds(off, 16)], None, None, None]
                for k in range(16):
                    a = k % 4
                    term = wks[k] * buf[k, pl.ds(off, 16)]
                    accs[a] = term if accs[a] is None else accs[a] + term
                for k2 in range(K - 16):
                    accs[k2] = accs[k2] + wks[16 + k2] * b2[
                        pl.ds(k2 * CW + off, 16)]
                acc = (accs[0] + accs[1]) + (accs[2] + accs[3])
                p = pm[pl.ds(off, 16)]
                return (bv + p * acc, av + p,
                        jnp.minimum(mnv, acc), jnp.maximum(mxv, acc))

            init = (zero16, zero16,
                    jnp.full((16,), jnp.inf, jnp.float32),
                    jnp.full((16,), -jnp.inf, jnp.float32))
            bv, av, mnv, mxv = lax.fori_loop(0, NB, body, init)

            @pl.when(c == C - 1)
            def _tail():
                tail_cp = pltpu.async_copy(tslab_hbm.at[gi], tail_v, semt)
                t2_cps = [pltpu.async_copy(
                    tslab_hbm.at[iks[16 + k2], pl.ds(0, 128)],
                    t2_v.at[pl.ds(k2 * 128, 128)], semt)
                    for k2 in range(K - 16)]
                pltpu.sync_copy(pmf_hbm.at[pl.ds(t * VP + C * CW, 128)],
                                pmt_v)
                tail_cp.wait()
                for cp in t2_cps:
                    cp.wait()
                mt = zero16
                for k in range(16):
                    mt = mt + wks[k] * tail_v[k, pl.ds(0, 16)]
                for k2 in range(K - 16):
                    mt = mt + wks[16 + k2] * t2_v[pl.ds(k2 * 128, 16)]
                corr_v[pl.ds(0, 16)] = zero16
                pt0 = i0 - C * CW
                pt1 = i16c - C * CW
                it0 = (pt0 >= 0) & (pt0 < TAIL)
                it1 = m1 & (pt1 >= 0) & (pt1 < TAIL)
                plsc.store_scatter(corr_v, [jnp.where(it0, pt0, 0)], delta0,
                                   mask=it0)
                plsc.store_scatter(corr_v, [jnp.where(it1, pt1, 0)], delta1,
                                   mask=it1)
                mt2 = mt + corr_v[pl.ds(0, 16)]
                pmt = pmt_v[pl.ds(0, 16)]
                bs = jnp.sum(bv + pmt * mt2)
                as_ = jnp.sum(av + pmt)
                mns = jnp.min(jnp.minimum(mnv, mt2))
                mxs = jnp.max(jnp.maximum(mxv, mt2))
                ov = (jnp.where(lane == 0, bs, 0.0)
                      + jnp.where(lane == 1, mns, 0.0)
                      + jnp.where(lane == 2, mxs, 0.0)
                      + jnp.where(lane == 3, as_, 0.0))
                o_v[pl.ds(0, 16)] = ov
                pltpu.sync_copy(o_v, parts_hbm.at[pl.ds(j * 128, 128)])

            @pl.when(c != C - 1)
            def _notail():
                bs = jnp.sum(bv)
                as_ = jnp.sum(av)
                mns = jnp.min(mnv)
                mxs = jnp.max(mxv)
                ov = (jnp.where(lane == 0, bs, 0.0)
                      + jnp.where(lane == 1, mns, 0.0)
                      + jnp.where(lane == 2, mxs, 0.0)
                      + jnp.where(lane == 3, as_, 0.0))
                o_v[pl.ds(0, 16)] = ov
                pltpu.sync_copy(o_v, parts_hbm.at[pl.ds(j * 128, 128)])


@jax.jit
def _sc_gather(npmi, tslab, idxf, wf, pmf):
    mesh = plsc.VectorSubcoreMesh(core_axis_name="c", subcore_axis_name="s")
    dbl = lambda shape, dt: [pltpu.VMEM(shape, dt), pltpu.VMEM(shape, dt)]
    run = functools.partial(
        pl.kernel,
        out_type=jax.ShapeDtypeStruct((NT * 128,), jnp.float32),
        mesh=mesh,
        scratch_types=(
            dbl((16,), jnp.int32)
            + dbl((128,), jnp.int32)
            + dbl((128,), jnp.float32)
            + dbl((16, CW), jnp.float32)
            + dbl(((K - 16) * CW,), jnp.float32)
            + dbl((CW,), jnp.float32)
            + [
                pltpu.VMEM((CW,), jnp.float32),
                pltpu.VMEM((K * 128,), jnp.float32),
                pltpu.VMEM((16, 128), jnp.float32),
                pltpu.VMEM(((K - 16) * 128,), jnp.float32),
                pltpu.VMEM((128,), jnp.float32),
                pltpu.VMEM((128,), jnp.float32),
                pltpu.SemaphoreType.DMA,
                pltpu.SemaphoreType.DMA,
                pltpu.SemaphoreType.DMA,
                pltpu.SemaphoreType.DMA,
            ]
        ),
        compiler_params=pltpu.CompilerParams(use_tc_tiling_on_sc=True,
                                             needs_layout_passes=False),
    )(_sc_body)
    return run(npmi, tslab, idxf, wf, pmf)


def _fin_body(parts_ref, out_ref):
    p = parts_ref[...]
    lane = lax.broadcasted_iota(jnp.int32, (T, C * 128), 1) % 128
    b = jnp.sum(jnp.where(lane == 0, p, 0.0), axis=1, keepdims=True)
    mn = jnp.min(jnp.where(lane == 1, p, jnp.inf), axis=1, keepdims=True)
    mx = jnp.max(jnp.where(lane == 2, p, -jnp.inf), axis=1, keepdims=True)
    a = jnp.sum(jnp.where(lane == 3, p, 0.0), axis=1, keepdims=True)
    lt = a - (b - mn * a) / (mx - mn)
    out_ref[0, 0] = jnp.sum(lt)


@jax.jit
def _finalize(parts):
    return pl.pallas_call(
        _fin_body,
        out_shape=jax.ShapeDtypeStruct((1, 1), jnp.float32),
        out_specs=pl.BlockSpec(memory_space=pltpu.SMEM),
    )(parts)


def kernel(jacobian, npmi_matrix):
    beta = jacobian.T
    beta_p = jnp.pad(beta, ((0, 0), (0, VP - V)), constant_values=NEG)
    premul, idx128, w128 = _prep(beta_p)
    tslab = jnp.pad(lax.slice(npmi_matrix, (0, C * CW), (V, V)),
                    ((0, 0), (0, 128 - TAIL)))
    parts = _sc_gather(npmi_matrix, tslab, idx128.reshape(T * 128),
                       w128.reshape(T * 128), premul.reshape(T * VP))
    loss = _finalize(parts.reshape(T, C * 128))
    return loss[0, 0]
